# baseline (speedup 1.0000x reference)
import functools

import jax
import jax.numpy as jnp
from jax import lax
from jax.experimental import pallas as pl
from jax.experimental.pallas import tpu as pltpu
from jax.experimental.pallas import tpu_sc as plsc

_NC = 2
_NS = 16
_NW = _NC * _NS

_LANES = 128
_FPAD = 4
_BSUB = 8


def _sc_gather(table, idx, n_groups, n_slabs, groups_per_chunk):
    d = table.shape[1]
    idx_per_group = n_slabs * _BSUB * _FPAD
    w_groups = n_groups // _NW
    per_w = w_groups * idx_per_group
    chunk = groups_per_chunk * idx_per_group
    n_chunks = w_groups // groups_per_chunk
    assert n_groups % _NW == 0 and w_groups % groups_per_chunk == 0
    assert n_chunks % 2 == 0

    mesh = plsc.VectorSubcoreMesh(core_axis_name="c", subcore_axis_name="s")

    @functools.partial(
        pl.kernel,
        mesh=mesh,
        out_type=jax.ShapeDtypeStruct((n_groups * idx_per_group, d), jnp.float32),
        scratch_types=[
            pltpu.VMEM((per_w,), jnp.int32),
            pltpu.VMEM((2, chunk, d), jnp.float32),
            pltpu.SemaphoreType.DMA,
            pltpu.SemaphoreType.DMA,
        ],
        compiler_params=pltpu.CompilerParams(use_tc_tiling_on_sc=False),
    )
    def gather_kernel(table_hbm, idx_hbm, out_hbm, idx_v, rows_v, sem0, sem1):
        wid = lax.axis_index("s") * _NC + lax.axis_index("c")
        base = wid * per_w
        pltpu.sync_copy(idx_hbm.at[pl.ds(base, per_w)], idx_v)

        sems = (sem0, sem1)

        def start(c, slot):
            pltpu.async_copy(
                table_hbm.at[idx_v.at[pl.ds(c * chunk, chunk)]],
                rows_v.at[slot],
                sems[slot],
            )

        def drain(c, slot):
            pltpu.make_async_copy(
                table_hbm.at[idx_v.at[pl.ds(c * chunk, chunk)]],
                rows_v.at[slot],
                sems[slot],
            ).wait()
            pltpu.sync_copy(
                rows_v.at[slot],
                out_hbm.at[pl.ds(base + c * chunk, chunk)],
            )

        start(0, 0)

        def body(cc, carry):
            c0 = 2 * cc
            start(c0 + 1, 1)
            drain(c0, 0)

            @pl.when(c0 + 2 < n_chunks)
            def _():
                start(c0 + 2, 0)

            drain(c0 + 1, 1)
            return carry

        lax.fori_loop(0, n_chunks // 2, body, 0)

    return gather_kernel(table, idx)


def _tc_mlp(emb4, w1p, b1, w2, b2, w3, block_g):
    n_groups, n_slabs, _, _ = emb4.shape
    h1 = w1p.shape[2]
    h2 = w2.shape[1]
    out = w3.shape[1]
    batch = n_groups * _BSUB
    block_m = block_g * _BSUB

    def body(x_ref, w1_ref, b1_ref, w2_ref, b2_ref, w3_ref, o_ref):
        acc = jnp.zeros((block_m, h1), jnp.float32)
        for jt in range(n_slabs):
            xj = x_ref[:, jt].reshape(block_m, _LANES)
            acc = acc + jnp.dot(
                xj, w1_ref[jt], preferred_element_type=jnp.float32
            )
        x = jnp.maximum(acc + b1_ref[...], 0.0)
        x = jnp.dot(x, w2_ref[...], preferred_element_type=jnp.float32)
        x = jnp.maximum(x + b2_ref[...], 0.0)
        o_ref[...] = jnp.dot(x, w3_ref[...], preferred_element_type=jnp.float32)

    return pl.pallas_call(
        body,
        grid=(n_groups // block_g,),
        in_specs=[
            pl.BlockSpec((block_g, n_slabs, _BSUB, _LANES), lambda i: (i, 0, 0, 0)),
            pl.BlockSpec((n_slabs, _LANES, h1), lambda i: (0, 0, 0)),
            pl.BlockSpec((1, h1), lambda i: (0, 0)),
            pl.BlockSpec((h1, h2), lambda i: (0, 0)),
            pl.BlockSpec((1, h2), lambda i: (0, 0)),
            pl.BlockSpec((h2, out), lambda i: (0, 0)),
        ],
        out_specs=pl.BlockSpec((block_m, out), lambda i: (i, 0)),
        out_shape=jax.ShapeDtypeStruct((batch, out), jnp.float32),
    )(emb4, w1p, b1, w2, b2, w3)


def kernel(X, table, W1, b1, W2, b2, W3, b3):
    batch, n_fields = X.shape
    d = table.shape[1]
    in_dim = n_fields * d
    h1 = W1.shape[1]
    n_slabs = -(-n_fields // _FPAD)
    f_pad = n_slabs * _FPAD
    n_groups = batch // _BSUB

    xp = jnp.pad(X.astype(jnp.int32), ((0, 0), (0, f_pad - n_fields)))
    idx = (
        xp.reshape(n_groups, _BSUB, n_slabs, _FPAD)
        .transpose(0, 2, 1, 3)
        .reshape(-1)
    )

    emb2 = _sc_gather(table, idx, n_groups, n_slabs, groups_per_chunk=4)
    emb4 = emb2.reshape(n_groups, n_slabs, _BSUB, _LANES)

    w1p = jnp.concatenate(
        [W1, jnp.zeros((n_slabs * _LANES - in_dim, h1), W1.dtype)], axis=0
    ).reshape(n_slabs, _LANES, h1)

    y = _tc_mlp(
        emb4, w1p, b1.reshape(1, -1), W2, b2.reshape(1, -1), W3, block_g=128
    )
    return y + b3[None, :]


# speedup vs baseline: 1.6814x; 1.6814x over previous
import functools

import jax
import jax.numpy as jnp
from jax import lax
from jax.experimental import pallas as pl
from jax.experimental.pallas import tpu as pltpu
from jax.experimental.pallas import tpu_sc as plsc

_NC = 2
_NS = 16
_NW = _NC * _NS

_LANES = 128
_FPAD = 4
_BSUB = 8
_VL = 16


def _sc_gather(table, x_idx, n_slabs, chunk):
    batch, n_fields = x_idx.shape
    d = table.shape[1]
    f_pad = n_slabs * _FPAD
    samples_w = batch // _NW
    per_w = samples_w * f_pad
    n_chunks = per_w // chunk
    n_vecs = per_w // _VL
    assert batch % _NW == 0 and per_w % chunk == 0 and n_chunks % 2 == 0

    mesh = plsc.VectorSubcoreMesh(core_axis_name="c", subcore_axis_name="s")

    @functools.partial(
        pl.kernel,
        mesh=mesh,
        out_type=jax.ShapeDtypeStruct((batch * f_pad, d), jnp.float32),
        scratch_types=[
            pltpu.VMEM((samples_w, n_fields), jnp.int32),
            pltpu.VMEM((per_w,), jnp.int32),
            pltpu.VMEM((2, chunk, d), jnp.float32),
            pltpu.SemaphoreType.DMA,
            pltpu.SemaphoreType.DMA,
        ],
        compiler_params=pltpu.CompilerParams(
            use_tc_tiling_on_sc=False, needs_layout_passes=False
        ),
    )
    def gather_kernel(table_hbm, x_hbm, out_hbm, xv, idx_v, rows_v, sem0, sem1):
        wid = lax.axis_index("s") * _NC + lax.axis_index("c")
        base = wid * per_w
        s_base = wid * samples_w
        pltpu.sync_copy(x_hbm.at[pl.ds(s_base, samples_w)], xv)

        lane = lax.iota(jnp.int32, _VL)
        lane_bsub = jnp.bitwise_and(jnp.right_shift(lane, 2), 3)
        lane_fsub = jnp.bitwise_and(lane, 3)

        def permute_g(g, carry):
            def permute_jt(jt, carry2):
                def permute_half(half, carry3):
                    bsub = half * (_VL // _FPAD) + lane_bsub
                    s = g * _BSUB + bsub
                    f = jt * _FPAD + lane_fsub
                    f = jnp.where(f < n_fields, f, f - n_fields)
                    vals = plsc.load_gather(xv, [s, f])
                    off = ((g * n_slabs + jt) * (_BSUB * _FPAD)) + half * _VL
                    idx_v[pl.ds(off, _VL)] = vals
                    return carry3

                return lax.fori_loop(0, _BSUB * _FPAD // _VL, permute_half, carry2)

            return lax.fori_loop(0, n_slabs, permute_jt, carry)

        lax.fori_loop(0, samples_w // _BSUB, permute_g, 0)

        sems = (sem0, sem1)

        def start(c, slot):
            pltpu.async_copy(
                table_hbm.at[idx_v.at[pl.ds(c * chunk, chunk)]],
                rows_v.at[slot],
                sems[slot],
            )

        def drain(c, slot):
            pltpu.make_async_copy(
                table_hbm.at[idx_v.at[pl.ds(c * chunk, chunk)]],
                rows_v.at[slot],
                sems[slot],
            ).wait()
            pltpu.sync_copy(
                rows_v.at[slot],
                out_hbm.at[pl.ds(base + c * chunk, chunk)],
            )

        start(0, 0)

        def body(cc, carry):
            c0 = 2 * cc
            start(c0 + 1, 1)
            drain(c0, 0)

            @pl.when(c0 + 2 < n_chunks)
            def _():
                start(c0 + 2, 0)

            drain(c0 + 1, 1)
            return carry

        lax.fori_loop(0, n_chunks // 2, body, 0)

    return gather_kernel(table, x_idx)


def _tc_mlp(emb4, w1p, b1, w2, b2, w3, block_g):
    n_groups, n_slabs, _, _ = emb4.shape
    h1 = w1p.shape[2]
    h2 = w2.shape[1]
    out = w3.shape[1]
    batch = n_groups * _BSUB
    block_m = block_g * _BSUB

    def body(x_ref, w1_ref, b1_ref, w2_ref, b2_ref, w3_ref, o_ref):
        acc = jnp.zeros((block_m, h1), jnp.float32)
        for jt in range(n_slabs):
            xj = x_ref[:, jt].reshape(block_m, _LANES)
            acc = acc + jnp.dot(
                xj, w1_ref[jt], preferred_element_type=jnp.float32
            )
        x = jnp.maximum(acc + b1_ref[...], 0.0)
        x = jnp.dot(x, w2_ref[...], preferred_element_type=jnp.float32)
        x = jnp.maximum(x + b2_ref[...], 0.0)
        o_ref[...] = jnp.dot(x, w3_ref[...], preferred_element_type=jnp.float32)

    return pl.pallas_call(
        body,
        grid=(n_groups // block_g,),
        in_specs=[
            pl.BlockSpec((block_g, n_slabs, _BSUB, _LANES), lambda i: (i, 0, 0, 0)),
            pl.BlockSpec((n_slabs, _LANES, h1), lambda i: (0, 0, 0)),
            pl.BlockSpec((1, h1), lambda i: (0, 0)),
            pl.BlockSpec((h1, h2), lambda i: (0, 0)),
            pl.BlockSpec((1, h2), lambda i: (0, 0)),
            pl.BlockSpec((h2, out), lambda i: (0, 0)),
        ],
        out_specs=pl.BlockSpec((block_m, out), lambda i: (i, 0)),
        out_shape=jax.ShapeDtypeStruct((batch, out), jnp.float32),
    )(emb4, w1p, b1, w2, b2, w3)


def kernel(X, table, W1, b1, W2, b2, W3, b3):
    batch, n_fields = X.shape
    d = table.shape[1]
    in_dim = n_fields * d
    h1 = W1.shape[1]
    n_slabs = -(-n_fields // _FPAD)
    n_groups = batch // _BSUB

    emb2 = _sc_gather(table, X.astype(jnp.int32), n_slabs, chunk=1024)
    emb4 = emb2.reshape(n_groups, n_slabs, _BSUB, _LANES)

    w1p = jnp.concatenate(
        [W1, jnp.zeros((n_slabs * _LANES - in_dim, h1), W1.dtype)], axis=0
    ).reshape(n_slabs, _LANES, h1)

    y = _tc_mlp(
        emb4, w1p, b1.reshape(1, -1), W2, b2.reshape(1, -1), W3, block_g=128
    )
    return y + b3[None, :]


# speedup vs baseline: 2.1125x; 1.2564x over previous
import functools

import jax
import jax.numpy as jnp
from jax import lax
from jax.experimental import pallas as pl
from jax.experimental.pallas import tpu as pltpu
from jax.experimental.pallas import tpu_sc as plsc

_NC = 2
_NS = 16
_NW = _NC * _NS

_LANES = 128
_FPAD = 4
_BSUB = 8
_VL = 16


def _sc_gather(table, x_idx, n_slabs, chunk):
    batch, n_fields = x_idx.shape
    d = table.shape[1]
    f_pad = n_slabs * _FPAD
    samples_w = batch // _NW
    per_w = samples_w * f_pad
    n_chunks = per_w // chunk
    n_vecs = per_w // _VL
    assert batch % _NW == 0 and per_w % chunk == 0 and n_chunks % 2 == 0

    mesh = plsc.VectorSubcoreMesh(core_axis_name="c", subcore_axis_name="s")

    @functools.partial(
        pl.kernel,
        mesh=mesh,
        out_type=jax.ShapeDtypeStruct((batch * f_pad, d), jnp.float32),
        scratch_types=[
            pltpu.VMEM((samples_w, n_fields), jnp.int32),
            pltpu.VMEM((per_w,), jnp.int32),
            pltpu.VMEM((2, chunk, d), jnp.float32),
            pltpu.SemaphoreType.DMA,
            pltpu.SemaphoreType.DMA,
        ],
        compiler_params=pltpu.CompilerParams(
            use_tc_tiling_on_sc=False, needs_layout_passes=False
        ),
    )
    def gather_kernel(table_hbm, x_hbm, out_hbm, xv, idx_v, rows_v, sem0, sem1):
        wid = lax.axis_index("s") * _NC + lax.axis_index("c")
        base = wid * per_w
        s_base = wid * samples_w
        pltpu.sync_copy(x_hbm.at[pl.ds(s_base, samples_w)], xv)

        lane = lax.iota(jnp.int32, _VL)
        lane_bsub = jnp.bitwise_and(jnp.right_shift(lane, 2), 3)
        lane_fsub = jnp.bitwise_and(lane, 3)

        def permute_g(g, carry):
            def permute_jt(jt, carry2):
                def permute_half(half, carry3):
                    bsub = half * (_VL // _FPAD) + lane_bsub
                    s = g * _BSUB + bsub
                    f = jt * _FPAD + lane_fsub
                    f = jnp.where(f < n_fields, f, f - n_fields)
                    vals = plsc.load_gather(xv, [s, f])
                    off = ((g * n_slabs + jt) * (_BSUB * _FPAD)) + half * _VL
                    idx_v[pl.ds(off, _VL)] = vals
                    return carry3

                return lax.fori_loop(0, _BSUB * _FPAD // _VL, permute_half, carry2)

            return lax.fori_loop(0, n_slabs, permute_jt, carry)

        lax.fori_loop(0, samples_w // _BSUB, permute_g, 0)

        sems = (sem0, sem1)

        def start(c, slot):
            pltpu.async_copy(
                table_hbm.at[idx_v.at[pl.ds(c * chunk, chunk)]],
                rows_v.at[slot],
                sems[slot],
            )

        def drain(c, slot):
            pltpu.make_async_copy(
                table_hbm.at[idx_v.at[pl.ds(c * chunk, chunk)]],
                rows_v.at[slot],
                sems[slot],
            ).wait()
            pltpu.sync_copy(
                rows_v.at[slot],
                out_hbm.at[pl.ds(base + c * chunk, chunk)],
            )

        start(0, 0)

        def body(cc, carry):
            c0 = 2 * cc
            start(c0 + 1, 1)
            drain(c0, 0)

            @pl.when(c0 + 2 < n_chunks)
            def _():
                start(c0 + 2, 0)

            drain(c0 + 1, 1)
            return carry

        lax.fori_loop(0, n_chunks // 2, body, 0)

    return gather_kernel(table, x_idx)


def _tc_transpose_table(table_t, block_n):
    d, vocab = table_t.shape
    out_rows = vocab * d // _LANES
    rows_per_block = block_n * d // _LANES
    n_blocks = -(-vocab // block_n)

    pack = _LANES // d

    def body(x_ref, o_ref):
        xt = x_ref[...].T
        xt3 = xt.reshape(rows_per_block, pack, d)
        for s in range(pack):
            o_ref[:, s * d : (s + 1) * d] = xt3[:, s, :]

    return pl.pallas_call(
        body,
        grid=(n_blocks,),
        in_specs=[pl.BlockSpec((d, block_n), lambda i: (0, i))],
        out_specs=pl.BlockSpec((rows_per_block, _LANES), lambda i: (i, 0)),
        out_shape=jax.ShapeDtypeStruct((out_rows, _LANES), jnp.float32),
    )(table_t)


def _tc_mlp(emb4, w1p, b1, w2, b2, w3, block_g):
    n_groups, n_slabs, _, _ = emb4.shape
    h1 = w1p.shape[2]
    h2 = w2.shape[1]
    out = w3.shape[1]
    batch = n_groups * _BSUB
    block_m = block_g * _BSUB

    def body(x_ref, w1_ref, b1_ref, w2_ref, b2_ref, w3_ref, o_ref):
        acc = jnp.zeros((block_m, h1), jnp.float32)
        for jt in range(n_slabs):
            xj = x_ref[:, jt].reshape(block_m, _LANES)
            acc = acc + jnp.dot(
                xj, w1_ref[jt], preferred_element_type=jnp.float32
            )
        x = jnp.maximum(acc + b1_ref[...], 0.0)
        x = jnp.dot(x, w2_ref[...], preferred_element_type=jnp.float32)
        x = jnp.maximum(x + b2_ref[...], 0.0)
        o_ref[...] = jnp.dot(x, w3_ref[...], preferred_element_type=jnp.float32)

    return pl.pallas_call(
        body,
        grid=(n_groups // block_g,),
        in_specs=[
            pl.BlockSpec((block_g, n_slabs, _BSUB, _LANES), lambda i: (i, 0, 0, 0)),
            pl.BlockSpec((n_slabs, _LANES, h1), lambda i: (0, 0, 0)),
            pl.BlockSpec((1, h1), lambda i: (0, 0)),
            pl.BlockSpec((h1, h2), lambda i: (0, 0)),
            pl.BlockSpec((1, h2), lambda i: (0, 0)),
            pl.BlockSpec((h2, out), lambda i: (0, 0)),
        ],
        out_specs=pl.BlockSpec((block_m, out), lambda i: (i, 0)),
        out_shape=jax.ShapeDtypeStruct((batch, out), jnp.float32),
    )(emb4, w1p, b1, w2, b2, w3)


def kernel(X, table, W1, b1, W2, b2, W3, b3):
    batch, n_fields = X.shape
    d = table.shape[1]
    in_dim = n_fields * d
    h1 = W1.shape[1]
    n_slabs = -(-n_fields // _FPAD)
    n_groups = batch // _BSUB

    table_lin = _tc_transpose_table(table.T, block_n=8192)
    table_rm = table_lin.reshape(table.shape[0], d)

    emb2 = _sc_gather(table_rm, X.astype(jnp.int32), n_slabs, chunk=1024)
    emb4 = emb2.reshape(n_groups, n_slabs, _BSUB, _LANES)

    w1p = jnp.concatenate(
        [W1, jnp.zeros((n_slabs * _LANES - in_dim, h1), W1.dtype)], axis=0
    ).reshape(n_slabs, _LANES, h1)

    y = _tc_mlp(
        emb4, w1p, b1.reshape(1, -1), W2, b2.reshape(1, -1), W3, block_g=128
    )
    return y + b3[None, :]


# speedup vs baseline: 2.1604x; 1.0227x over previous
import functools

import jax
import jax.numpy as jnp
from jax import lax
from jax.experimental import pallas as pl
from jax.experimental.pallas import tpu as pltpu
from jax.experimental.pallas import tpu_sc as plsc

_NC = 2
_NS = 16
_NW = _NC * _NS

_LANES = 128
_FPAD = 4
_BSUB = 8
_VL = 16


def _sc_gather(table, x_idx, n_slabs, chunk):
    batch, n_fields = x_idx.shape
    d = table.shape[1]
    f_pad = n_slabs * _FPAD
    samples_w = batch // _NW
    per_w = samples_w * f_pad
    n_chunks = per_w // chunk
    n_vecs = per_w // _VL
    assert batch % _NW == 0 and per_w % chunk == 0 and n_chunks % 2 == 0

    mesh = plsc.VectorSubcoreMesh(core_axis_name="c", subcore_axis_name="s")

    @functools.partial(
        pl.kernel,
        mesh=mesh,
        out_type=jax.ShapeDtypeStruct((batch * f_pad, d), jnp.float32),
        scratch_types=[
            pltpu.VMEM((samples_w, n_fields), jnp.int32),
            pltpu.VMEM((per_w,), jnp.int32),
            pltpu.VMEM((2, chunk, d), jnp.float32),
            pltpu.SemaphoreType.DMA,
            pltpu.SemaphoreType.DMA,
        ],
        compiler_params=pltpu.CompilerParams(
            use_tc_tiling_on_sc=False, needs_layout_passes=False
        ),
    )
    def gather_kernel(table_hbm, x_hbm, out_hbm, xv, idx_v, rows_v, sem0, sem1):
        wid = lax.axis_index("s") * _NC + lax.axis_index("c")
        base = wid * per_w
        s_base = wid * samples_w
        pltpu.sync_copy(x_hbm.at[pl.ds(s_base, samples_w)], xv)

        lane = lax.iota(jnp.int32, _VL)
        lane_bsub = jnp.bitwise_and(jnp.right_shift(lane, 2), 3)
        lane_fsub = jnp.bitwise_and(lane, 3)

        def permute_g(g, carry):
            def permute_jt(jt, carry2):
                def permute_half(half, carry3):
                    bsub = half * (_VL // _FPAD) + lane_bsub
                    s = g * _BSUB + bsub
                    f = jt * _FPAD + lane_fsub
                    f = jnp.where(f < n_fields, f, f - n_fields)
                    vals = plsc.load_gather(xv, [s, f])
                    off = ((g * n_slabs + jt) * (_BSUB * _FPAD)) + half * _VL
                    idx_v[pl.ds(off, _VL)] = vals
                    return carry3

                return lax.fori_loop(0, _BSUB * _FPAD // _VL, permute_half, carry2)

            return lax.fori_loop(0, n_slabs, permute_jt, carry)

        lax.fori_loop(0, samples_w // _BSUB, permute_g, 0)

        sems = (sem0, sem1)

        def start(c, slot):
            pltpu.async_copy(
                table_hbm.at[idx_v.at[pl.ds(c * chunk, chunk)]],
                rows_v.at[slot],
                sems[slot],
            )

        def drain(c, slot):
            pltpu.make_async_copy(
                table_hbm.at[idx_v.at[pl.ds(c * chunk, chunk)]],
                rows_v.at[slot],
                sems[slot],
            ).wait()
            pltpu.sync_copy(
                rows_v.at[slot],
                out_hbm.at[pl.ds(base + c * chunk, chunk)],
            )

        start(0, 0)

        def body(cc, carry):
            c0 = 2 * cc
            start(c0 + 1, 1)
            drain(c0, 0)

            @pl.when(c0 + 2 < n_chunks)
            def _():
                start(c0 + 2, 0)

            drain(c0 + 1, 1)
            return carry

        lax.fori_loop(0, n_chunks // 2, body, 0)

    return gather_kernel(table, x_idx)


def _tc_transpose_table(table_t, block_n):
    d, vocab = table_t.shape
    out_rows = vocab * d // _LANES
    rows_per_block = block_n * d // _LANES
    n_blocks = -(-vocab // block_n)

    pack = _LANES // d

    def body(x_ref, o_ref):
        xt = x_ref[...].T
        xt3 = xt.reshape(rows_per_block, pack, d)
        for s in range(pack):
            o_ref[:, s * d : (s + 1) * d] = xt3[:, s, :]

    return pl.pallas_call(
        body,
        grid=(n_blocks,),
        in_specs=[pl.BlockSpec((d, block_n), lambda i: (0, i))],
        out_specs=pl.BlockSpec((rows_per_block, _LANES), lambda i: (i, 0)),
        out_shape=jax.ShapeDtypeStruct((out_rows, _LANES), jnp.float32),
        compiler_params=pltpu.CompilerParams(fuse_transposed_lhs_in_matmul=True),
    )(table_t)


def _tc_mlp(emb4, w1p, b1, w2, b2, w3, block_g):
    n_groups, n_slabs, _, _ = emb4.shape
    h1 = w1p.shape[2]
    h2 = w2.shape[1]
    out = w3.shape[1]
    batch = n_groups * _BSUB
    block_m = block_g * _BSUB

    def body(x_ref, w1_ref, b1_ref, w2_ref, b2_ref, w3_ref, o_ref):
        acc = jnp.zeros((block_m, h1), jnp.float32)
        for jt in range(n_slabs):
            xj = x_ref[:, jt].reshape(block_m, _LANES)
            acc = acc + jnp.dot(
                xj, w1_ref[jt], preferred_element_type=jnp.float32
            )
        x = jnp.maximum(acc + b1_ref[...], 0.0)
        x = jnp.dot(x, w2_ref[...], preferred_element_type=jnp.float32)
        x = jnp.maximum(x + b2_ref[...], 0.0)
        o_ref[...] = jnp.dot(x, w3_ref[...], preferred_element_type=jnp.float32)

    return pl.pallas_call(
        body,
        grid=(n_groups // block_g,),
        in_specs=[
            pl.BlockSpec((block_g, n_slabs, _BSUB, _LANES), lambda i: (i, 0, 0, 0)),
            pl.BlockSpec((n_slabs, _LANES, h1), lambda i: (0, 0, 0)),
            pl.BlockSpec((1, h1), lambda i: (0, 0)),
            pl.BlockSpec((h1, h2), lambda i: (0, 0)),
            pl.BlockSpec((1, h2), lambda i: (0, 0)),
            pl.BlockSpec((h2, out), lambda i: (0, 0)),
        ],
        out_specs=pl.BlockSpec((block_m, out), lambda i: (i, 0)),
        out_shape=jax.ShapeDtypeStruct((batch, out), jnp.float32),
    )(emb4, w1p, b1, w2, b2, w3)


def kernel(X, table, W1, b1, W2, b2, W3, b3):
    batch, n_fields = X.shape
    d = table.shape[1]
    in_dim = n_fields * d
    h1 = W1.shape[1]
    n_slabs = -(-n_fields // _FPAD)
    n_groups = batch // _BSUB

    table_lin = _tc_transpose_table(table.T, block_n=16384)
    table_rm = table_lin.reshape(table.shape[0], d)

    emb2 = _sc_gather(table_rm, X.astype(jnp.int32), n_slabs, chunk=1024)
    emb4 = emb2.reshape(n_groups, n_slabs, _BSUB, _LANES)

    w1p = jnp.concatenate(
        [W1, jnp.zeros((n_slabs * _LANES - in_dim, h1), W1.dtype)], axis=0
    ).reshape(n_slabs, _LANES, h1)

    y = _tc_mlp(
        emb4, w1p, b1.reshape(1, -1), W2, b2.reshape(1, -1), W3, block_g=128
    )
    return y + b3[None, :]


# speedup vs baseline: 2.1875x; 1.0126x over previous
import functools

import jax
import jax.numpy as jnp
from jax import lax
from jax.experimental import pallas as pl
from jax.experimental.pallas import tpu as pltpu
from jax.experimental.pallas import tpu_sc as plsc

_NC = 2
_NS = 16
_NW = _NC * _NS

_LANES = 128
_FPAD = 4
_BSUB = 8
_VL = 16


def _sc_gather(table, x_idx, n_slabs, chunk):
    batch, n_fields = x_idx.shape
    d = table.shape[1]
    f_pad = n_slabs * _FPAD
    samples_w = batch // _NW
    per_w = samples_w * f_pad
    n_chunks = per_w // chunk
    n_vecs = per_w // _VL
    assert batch % _NW == 0 and per_w % chunk == 0 and n_chunks % 2 == 0

    mesh = plsc.VectorSubcoreMesh(core_axis_name="c", subcore_axis_name="s")

    @functools.partial(
        pl.kernel,
        mesh=mesh,
        out_type=jax.ShapeDtypeStruct((batch * f_pad, d), jnp.float32),
        scratch_types=[
            pltpu.VMEM((samples_w, n_fields), jnp.int32),
            pltpu.VMEM((per_w,), jnp.int32),
            pltpu.VMEM((2, chunk, d), jnp.float32),
            pltpu.SemaphoreType.DMA,
            pltpu.SemaphoreType.DMA,
        ],
        compiler_params=pltpu.CompilerParams(
            use_tc_tiling_on_sc=False, needs_layout_passes=False
        ),
    )
    def gather_kernel(table_hbm, x_hbm, out_hbm, xv, idx_v, rows_v, sem0, sem1):
        wid = lax.axis_index("s") * _NC + lax.axis_index("c")
        base = wid * per_w
        s_base = wid * samples_w
        pltpu.sync_copy(x_hbm.at[pl.ds(s_base, samples_w)], xv)

        lane = lax.iota(jnp.int32, _VL)
        lane_bsub = jnp.bitwise_and(jnp.right_shift(lane, 2), 3)
        lane_fsub = jnp.bitwise_and(lane, 3)

        def permute_g(g, carry):
            def permute_jt(jt, carry2):
                def permute_half(half, carry3):
                    bsub = half * (_VL // _FPAD) + lane_bsub
                    s = g * _BSUB + bsub
                    f = jt * _FPAD + lane_fsub
                    f = jnp.where(f < n_fields, f, f - n_fields)
                    vals = plsc.load_gather(xv, [s, f])
                    off = ((g * n_slabs + jt) * (_BSUB * _FPAD)) + half * _VL
                    idx_v[pl.ds(off, _VL)] = vals
                    return carry3

                return lax.fori_loop(0, _BSUB * _FPAD // _VL, permute_half, carry2)

            return lax.fori_loop(0, n_slabs, permute_jt, carry)

        lax.fori_loop(0, samples_w // _BSUB, permute_g, 0)

        sems = (sem0, sem1)

        def start(c, slot):
            pltpu.async_copy(
                table_hbm.at[idx_v.at[pl.ds(c * chunk, chunk)]],
                rows_v.at[slot],
                sems[slot],
            )

        def drain(c, slot):
            pltpu.make_async_copy(
                table_hbm.at[idx_v.at[pl.ds(c * chunk, chunk)]],
                rows_v.at[slot],
                sems[slot],
            ).wait()
            pltpu.sync_copy(
                rows_v.at[slot],
                out_hbm.at[pl.ds(base + c * chunk, chunk)],
            )

        start(0, 0)

        def body(cc, carry):
            c0 = 2 * cc
            start(c0 + 1, 1)
            drain(c0, 0)

            @pl.when(c0 + 2 < n_chunks)
            def _():
                start(c0 + 2, 0)

            drain(c0 + 1, 1)
            return carry

        lax.fori_loop(0, n_chunks // 2, body, 0)

    return gather_kernel(table, x_idx)


def _tc_transpose_table(table_t, block_n):
    d, vocab = table_t.shape
    out_rows = vocab * d // _LANES
    rows_per_block = block_n * d // _LANES
    n_blocks = -(-vocab // block_n)

    pack = _LANES // d

    def body(x_ref, o_ref):
        xt = x_ref[...].T
        xt3 = xt.reshape(rows_per_block, pack, d)
        for s in range(pack):
            o_ref[:, s * d : (s + 1) * d] = xt3[:, s, :]

    return pl.pallas_call(
        body,
        grid=(n_blocks,),
        in_specs=[pl.BlockSpec((d, block_n), lambda i: (0, i))],
        out_specs=pl.BlockSpec((rows_per_block, _LANES), lambda i: (i, 0)),
        out_shape=jax.ShapeDtypeStruct((out_rows, _LANES), jnp.float32),
        compiler_params=pltpu.CompilerParams(fuse_transposed_lhs_in_matmul=True),
    )(table_t)


def _tc_mlp(emb4, w1p, b1, w2, b2, w3, b3, block_g):
    n_groups, n_slabs, _, _ = emb4.shape
    h1 = w1p.shape[2]
    h2 = w2.shape[1]
    out = w3.shape[1]
    batch = n_groups * _BSUB
    block_m = block_g * _BSUB

    def body(x_ref, w1_ref, b1_ref, w2_ref, b2_ref, w3_ref, b3_ref, o_ref):
        acc = jnp.zeros((block_m, h1), jnp.float32)
        for jt in range(n_slabs):
            xj = x_ref[:, jt].reshape(block_m, _LANES)
            acc = acc + jnp.dot(
                xj, w1_ref[jt], preferred_element_type=jnp.float32
            )
        x = jnp.maximum(acc + b1_ref[...], 0.0)
        x = jnp.dot(x, w2_ref[...], preferred_element_type=jnp.float32)
        x = jnp.maximum(x + b2_ref[...], 0.0)
        o_ref[...] = (
            jnp.dot(x, w3_ref[...], preferred_element_type=jnp.float32)
            + b3_ref[...]
        )

    return pl.pallas_call(
        body,
        grid=(n_groups // block_g,),
        in_specs=[
            pl.BlockSpec((block_g, n_slabs, _BSUB, _LANES), lambda i: (i, 0, 0, 0)),
            pl.BlockSpec((n_slabs, _LANES, h1), lambda i: (0, 0, 0)),
            pl.BlockSpec((1, h1), lambda i: (0, 0)),
            pl.BlockSpec((h1, h2), lambda i: (0, 0)),
            pl.BlockSpec((1, h2), lambda i: (0, 0)),
            pl.BlockSpec((h2, out), lambda i: (0, 0)),
            pl.BlockSpec((1, out), lambda i: (0, 0)),
        ],
        out_specs=pl.BlockSpec((block_m, out), lambda i: (i, 0)),
        out_shape=jax.ShapeDtypeStruct((batch, out), jnp.float32),
    )(emb4, w1p, b1, w2, b2, w3, b3)


def kernel(X, table, W1, b1, W2, b2, W3, b3):
    batch, n_fields = X.shape
    d = table.shape[1]
    in_dim = n_fields * d
    h1 = W1.shape[1]
    n_slabs = -(-n_fields // _FPAD)
    n_groups = batch // _BSUB

    table_lin = _tc_transpose_table(table.T, block_n=16384)
    table_rm = table_lin.reshape(table.shape[0], d)

    emb2 = _sc_gather(table_rm, X.astype(jnp.int32), n_slabs, chunk=1024)
    emb4 = emb2.reshape(n_groups, n_slabs, _BSUB, _LANES)

    w1p = jnp.concatenate(
        [W1, jnp.zeros((n_slabs * _LANES - in_dim, h1), W1.dtype)], axis=0
    ).reshape(n_slabs, _LANES, h1)

    return _tc_mlp(
        emb4,
        w1p,
        b1.reshape(1, -1),
        W2,
        b2.reshape(1, -1),
        W3,
        b3.reshape(1, -1),
        block_g=128,
    )
